# baseline (speedup 1.0000x reference)
import functools
import math

import jax
import jax.numpy as jnp
from jax import lax
from jax.experimental import pallas as pl
from jax.experimental.pallas import tpu as pltpu
from jax.experimental.pallas import tpu_sc as plsc

_R = 128
_NF = 8


def _attn_topk_block(vs_ref, ve_ref, dead_ref, wq_ref, wk_ref, wv_ref,
                     wm_ref, bm_ref, outm_ref, idx_ref, u_ref):
    R, N = dead_ref.shape
    H = vs_ref.shape[1]
    f32 = jnp.float32

    vs = vs_ref[...]
    q = jnp.dot(vs, wq_ref[...], preferred_element_type=f32)

    k2 = jnp.dot(ve_ref[...], wk_ref[...], preferred_element_type=f32)
    G = 8
    parts = []
    for s0 in range(0, R, G):
        cf = lax.dot_general(q[s0:s0 + G], k2[s0 * N:(s0 + G) * N],
                             (((1,), (1,)), ((), ())),
                             preferred_element_type=f32)
        parts.extend(cf[r:r + 1, r * N:(r + 1) * N] for r in range(G))
    compat = jnp.concatenate(parts, axis=0)
    compat = compat * (1.0 / math.sqrt(H))

    dead = dead_ref[...] != 0
    c2 = jnp.where(dead, -1e30, compat)
    mx = jnp.max(c2, axis=1, keepdims=True)
    e = jnp.where(dead, 0.0, jnp.exp(c2 - mx))
    s = jnp.sum(e, axis=1, keepdims=True)
    score = jnp.where(s > 0.0, e / s, 0.0)

    for r in range(R):
        u_ref[r:r + 1, :] = jnp.dot(score[r:r + 1],
                                    ve_ref[r * N:(r + 1) * N, :],
                                    preferred_element_type=f32)
    va = jnp.dot(u_ref[...], wv_ref[...], preferred_element_type=f32)
    vm = (jnp.dot(vs, wm_ref[0:H, :], preferred_element_type=f32)
          + jnp.dot(va, wm_ref[H:2 * H, :], preferred_element_type=f32)
          + bm_ref[...])
    outm_ref[...] = jnp.maximum(vm, 0.0)

    iota = lax.broadcasted_iota(jnp.int32, (R, N), 1)
    row0 = pl.program_id(0) * R + lax.broadcasted_iota(jnp.int32, (R, 1), 0)
    cur = score
    for j in range(_NF):
        mj = jnp.max(cur, axis=1, keepdims=True)
        eq = cur == mj
        first = jnp.min(jnp.where(eq, iota, N), axis=1, keepdims=True)
        idx_ref[:, j:j + 1] = row0 * N + first
        cur = jnp.where(iota == first, -1.0, cur)


def _final_mlp_block(vs_ref, g_ref, wf_ref, bf_ref, outc_ref):
    H = vs_ref.shape[1]
    f32 = jnp.float32
    acc = (jnp.dot(vs_ref[...], wf_ref[0:H, :], preferred_element_type=f32)
           + jnp.dot(g_ref[...], wf_ref[H:, :], preferred_element_type=f32)
           + bf_ref[...])
    outc_ref[...] = jnp.maximum(acc, 0.0)


def _sc_gather(table, idx, n_rows, d):
    info = plsc.get_sparse_core_info()
    nw = info.num_cores * info.num_subcores
    b_per_w = n_rows // nw

    mesh = plsc.VectorSubcoreMesh(core_axis_name="c", subcore_axis_name="s")

    @functools.partial(
        pl.kernel, mesh=mesh,
        out_type=jax.ShapeDtypeStruct((n_rows, d), jnp.float32),
        scratch_types=[
            pltpu.VMEM((b_per_w,), jnp.int32),
            pltpu.VMEM((b_per_w, d), jnp.float32),
            pltpu.SemaphoreType.DMA,
        ],
    )
    def k(table_hbm, idx_hbm, out_hbm, idx_v, rows_v, sem):
        wid = lax.axis_index("s") * info.num_cores + lax.axis_index("c")
        base = wid * b_per_w
        pltpu.sync_copy(idx_hbm.at[pl.ds(base, b_per_w)], idx_v)
        pltpu.async_copy(table_hbm.at[idx_v], rows_v, sem).wait()
        pltpu.sync_copy(rows_v, out_hbm.at[pl.ds(base, b_per_w)])

    return k(table, idx)


def kernel(vs, ve, ve_dead, Wq, Wk, Wv, Wm, bm, Wf, bf):
    B, A, N, H = ve.shape
    BA = B * A
    R = _R
    vs2 = vs.reshape(BA, H)
    ve2 = ve.reshape(BA * N, H)
    dead2 = ve_dead.reshape(BA, N)

    outm, idx = pl.pallas_call(
        _attn_topk_block,
        grid=(BA // R,),
        in_specs=[
            pl.BlockSpec((R, H), lambda i: (i, 0)),
            pl.BlockSpec((R * N, H), lambda i: (i, 0)),
            pl.BlockSpec((R, N), lambda i: (i, 0)),
            pl.BlockSpec((H, H), lambda i: (0, 0)),
            pl.BlockSpec((H, H), lambda i: (0, 0)),
            pl.BlockSpec((H, H), lambda i: (0, 0)),
            pl.BlockSpec((2 * H, H), lambda i: (0, 0)),
            pl.BlockSpec((1, H), lambda i: (0, 0)),
        ],
        out_specs=[
            pl.BlockSpec((R, H), lambda i: (i, 0)),
            pl.BlockSpec((R, _NF), lambda i: (i, 0)),
        ],
        out_shape=[
            jax.ShapeDtypeStruct((BA, H), jnp.float32),
            jax.ShapeDtypeStruct((BA, _NF), jnp.int32),
        ],
        scratch_shapes=[
            pltpu.VMEM((R, H), jnp.float32),
        ],
    )(vs2, ve2, dead2, Wq, Wk, Wv, Wm, bm.reshape(1, H))

    gathered = _sc_gather(ve2, idx.reshape(BA * _NF), BA * _NF, H)
    gflat = gathered.reshape(BA, _NF * H)

    outc = pl.pallas_call(
        _final_mlp_block,
        grid=(BA // R,),
        in_specs=[
            pl.BlockSpec((R, H), lambda i: (i, 0)),
            pl.BlockSpec((R, _NF * H), lambda i: (i, 0)),
            pl.BlockSpec(((_NF + 1) * H, H), lambda i: (0, 0)),
            pl.BlockSpec((1, H), lambda i: (0, 0)),
        ],
        out_specs=pl.BlockSpec((R, H), lambda i: (i, 0)),
        out_shape=jax.ShapeDtypeStruct((BA, H), jnp.float32),
    )(vs2, gflat, Wf, bf.reshape(1, H))

    return outc.reshape(B, A, H), outm.reshape(B, A, H)


# speedup vs baseline: 1.0023x; 1.0023x over previous
import functools
import math

import jax
import jax.numpy as jnp
from jax import lax
from jax.experimental import pallas as pl
from jax.experimental.pallas import tpu as pltpu
from jax.experimental.pallas import tpu_sc as plsc

_R = 128
_NF = 8


def _attn_topk_block(vs_ref, ve_ref, dead_ref, wq_ref, wk_ref,
                     u_ref, idx_ref):
    R, N = dead_ref.shape
    H = vs_ref.shape[1]
    f32 = jnp.float32

    vs = vs_ref[...]
    q = jnp.dot(vs, wq_ref[...], preferred_element_type=f32)

    k2 = jnp.dot(ve_ref[...], wk_ref[...], preferred_element_type=f32)
    G = 8
    parts = []
    for s0 in range(0, R, G):
        cf = lax.dot_general(q[s0:s0 + G], k2[s0 * N:(s0 + G) * N],
                             (((1,), (1,)), ((), ())),
                             preferred_element_type=f32)
        parts.extend(cf[r:r + 1, r * N:(r + 1) * N] for r in range(G))
    compat = jnp.concatenate(parts, axis=0)
    compat = compat * (1.0 / math.sqrt(H))

    dead = dead_ref[...] != 0
    c2 = jnp.where(dead, -1e30, compat)
    mx = jnp.max(c2, axis=1, keepdims=True)
    e = jnp.where(dead, 0.0, jnp.exp(c2 - mx))
    s = jnp.sum(e, axis=1, keepdims=True)
    score = jnp.where(s > 0.0, e / s, 0.0)

    for r in range(R):
        u_ref[r:r + 1, :] = jnp.dot(score[r:r + 1],
                                    ve_ref[r * N:(r + 1) * N, :],
                                    preferred_element_type=f32)

    iota = lax.broadcasted_iota(jnp.int32, (R, N), 1)
    row0 = pl.program_id(0) * R + lax.broadcasted_iota(jnp.int32, (R, 1), 0)
    cur = score
    for j in range(_NF):
        mj = jnp.max(cur, axis=1, keepdims=True)
        eq = cur == mj
        first = jnp.min(jnp.where(eq, iota, N), axis=1, keepdims=True)
        idx_ref[:, j:j + 1] = row0 * N + first
        cur = jnp.where(iota == first, -1.0, cur)


def _final_mlp_block(vs_ref, u_ref, g_ref, wv_ref, wm_ref, bm_ref, wf_ref,
                     bf_ref, outc_ref, outm_ref):
    H = vs_ref.shape[1]
    f32 = jnp.float32
    vs = vs_ref[...]
    va = jnp.dot(u_ref[...], wv_ref[...], preferred_element_type=f32)
    vm = (jnp.dot(vs, wm_ref[0:H, :], preferred_element_type=f32)
          + jnp.dot(va, wm_ref[H:2 * H, :], preferred_element_type=f32)
          + bm_ref[...])
    outm_ref[...] = jnp.maximum(vm, 0.0)
    acc = (jnp.dot(vs, wf_ref[0:H, :], preferred_element_type=f32)
           + jnp.dot(g_ref[...], wf_ref[H:, :], preferred_element_type=f32)
           + bf_ref[...])
    outc_ref[...] = jnp.maximum(acc, 0.0)


def _sc_gather(table, idx, n_rows, d):
    info = plsc.get_sparse_core_info()
    nw = info.num_cores * info.num_subcores
    b_per_w = n_rows // nw

    mesh = plsc.VectorSubcoreMesh(core_axis_name="c", subcore_axis_name="s")

    @functools.partial(
        pl.kernel, mesh=mesh,
        out_type=jax.ShapeDtypeStruct((n_rows, d), jnp.float32),
        scratch_types=[
            pltpu.VMEM((b_per_w,), jnp.int32),
            pltpu.VMEM((b_per_w, d), jnp.float32),
            pltpu.SemaphoreType.DMA,
        ],
    )
    def k(table_hbm, idx_hbm, out_hbm, idx_v, rows_v, sem):
        wid = lax.axis_index("s") * info.num_cores + lax.axis_index("c")
        base = wid * b_per_w
        pltpu.sync_copy(idx_hbm.at[pl.ds(base, b_per_w)], idx_v)
        pltpu.async_copy(table_hbm.at[idx_v], rows_v, sem).wait()
        pltpu.sync_copy(rows_v, out_hbm.at[pl.ds(base, b_per_w)])

    return k(table, idx)


def kernel(vs, ve, ve_dead, Wq, Wk, Wv, Wm, bm, Wf, bf):
    B, A, N, H = ve.shape
    BA = B * A
    R = _R
    vs2 = vs.reshape(BA, H)
    ve2 = ve.reshape(BA * N, H)
    dead2 = ve_dead.reshape(BA, N)

    u, idx = pl.pallas_call(
        _attn_topk_block,
        grid=(BA // R,),
        in_specs=[
            pl.BlockSpec((R, H), lambda i: (i, 0)),
            pl.BlockSpec((R * N, H), lambda i: (i, 0)),
            pl.BlockSpec((R, N), lambda i: (i, 0)),
            pl.BlockSpec((H, H), lambda i: (0, 0)),
            pl.BlockSpec((H, H), lambda i: (0, 0)),
        ],
        out_specs=[
            pl.BlockSpec((R, H), lambda i: (i, 0)),
            pl.BlockSpec((R, _NF), lambda i: (i, 0)),
        ],
        out_shape=[
            jax.ShapeDtypeStruct((BA, H), jnp.float32),
            jax.ShapeDtypeStruct((BA, _NF), jnp.int32),
        ],
    )(vs2, ve2, dead2, Wq, Wk)

    gathered = _sc_gather(ve2, idx.reshape(BA * _NF), BA * _NF, H)
    gflat = gathered.reshape(BA, _NF * H)

    outc, outm = pl.pallas_call(
        _final_mlp_block,
        grid=(BA // R,),
        in_specs=[
            pl.BlockSpec((R, H), lambda i: (i, 0)),
            pl.BlockSpec((R, H), lambda i: (i, 0)),
            pl.BlockSpec((R, _NF * H), lambda i: (i, 0)),
            pl.BlockSpec((H, H), lambda i: (0, 0)),
            pl.BlockSpec((2 * H, H), lambda i: (0, 0)),
            pl.BlockSpec((1, H), lambda i: (0, 0)),
            pl.BlockSpec(((_NF + 1) * H, H), lambda i: (0, 0)),
            pl.BlockSpec((1, H), lambda i: (0, 0)),
        ],
        out_specs=[
            pl.BlockSpec((R, H), lambda i: (i, 0)),
            pl.BlockSpec((R, H), lambda i: (i, 0)),
        ],
        out_shape=[
            jax.ShapeDtypeStruct((BA, H), jnp.float32),
            jax.ShapeDtypeStruct((BA, H), jnp.float32),
        ],
    )(vs2, u, gflat, Wv, Wm, bm.reshape(1, H), Wf, bf.reshape(1, H))

    return outc.reshape(B, A, H), outm.reshape(B, A, H)
